# baseline (speedup 1.0000x reference)
import functools

import jax
import jax.numpy as jnp
from jax import lax
from jax.experimental import pallas as pl
from jax.experimental.pallas import tpu as pltpu
from jax.experimental.pallas import tpu_sc as plsc

N = 10000
E = 320000
NC = 2
NS = 16
PN = 10240
K = 128
G = 5
C0 = 40
C1 = 120
CMAX = max(C0, C1)
TOTCH = NS * (C0 + C1)
TOTCH_PAD = TOTCH + CMAX
EP = TOTCH_PAD * K
RPT = PN // NS
ZR = 128


def _mesh():
  return plsc.VectorSubcoreMesh(
      core_axis_name="c", subcore_axis_name="s",
      num_cores=NC, num_subcores=NS)


def _make_edge_agg(F):
  @functools.partial(
      pl.kernel,
      out_type=jax.ShapeDtypeStruct((NC, PN, F), jnp.float32),
      mesh=_mesh(),
      compiler_params=pltpu.CompilerParams(use_tc_tiling_on_sc=False),
      scratch_types=(
          [pltpu.VMEM((CMAX, K), jnp.int32)] * 2
          + [pltpu.VMEM((K, F), jnp.float32)] * (2 * G)
          + [pltpu.VMEM((ZR, F), jnp.float32)]
          + [pltpu.VMEM_SHARED((PN, F), jnp.float32)]
          + [pltpu.SemaphoreType.DMA] * 4
      ),
  )
  def agg(src_hbm, dst_hbm, hp_hbm, out_hbm, *refs):
    sidx, didx = refs[0], refs[1]
    buf_a = refs[2:2 + G]
    buf_b = refs[2 + G:2 + 2 * G]
    zbuf = refs[2 + 2 * G]
    acc = refs[3 + 2 * G]
    gs_a, gs_b, ss_a, ss_b = refs[4 + 2 * G:8 + 2 * G]

    c = lax.axis_index("c")
    s = lax.axis_index("s")
    start = jnp.where(c == 0, s * C0, NS * C0 + s * C1)
    ng = jnp.where(c == 0, C0 // G, C1 // G)
    np_ = ng // 2

    pltpu.sync_copy(src_hbm.at[pl.ds(start, CMAX)], sidx)
    pltpu.sync_copy(dst_hbm.at[pl.ds(start, CMAX)], didx)
    z = jnp.zeros((16,), jnp.float32)
    for i in range(ZR):
      for j in range(F // 16):
        zbuf[i, pl.ds(16 * j, 16)] = z
    for j in range(RPT // ZR):
      pltpu.sync_copy(zbuf, acc.at[pl.ds(s * RPT + j * ZR, ZR)])

    def fire_g(bufs, base, sem):
      for t in range(G):
        pltpu.async_copy(hp_hbm.at[sidx.at[base + t]], bufs[t], sem)

    def drain_g(bufs, sem):
      for t in range(G):
        pltpu.make_async_copy(hp_hbm.at[sidx.at[0]], bufs[t], sem).wait()

    def fire_s(bufs, base, sem):
      for t in range(G):
        pltpu.async_copy(bufs[t], acc.at[didx.at[base + t]], sem, add=True)

    def drain_s(bufs, sem):
      for t in range(G):
        pltpu.make_async_copy(bufs[t], acc.at[didx.at[0]], sem).wait()

    fire_g(buf_a, 0, gs_a)
    plsc.subcore_barrier()

    def pair(p, carry):
      a0 = (2 * p) * G
      b1 = (2 * p + 1) * G
      a2 = (2 * p + 2) * G
      fire_g(buf_b, b1, gs_b)
      drain_g(buf_a, gs_a)
      fire_s(buf_a, a0, ss_a)
      drain_s(buf_a, ss_a)
      fire_g(buf_a, a2, gs_a)
      drain_g(buf_b, gs_b)
      fire_s(buf_b, b1, ss_b)
      drain_s(buf_b, ss_b)
      return carry

    lax.fori_loop(0, np_ - 1, pair, 0)

    fire_g(buf_b, (ng - 1) * G, gs_b)
    drain_g(buf_a, gs_a)
    fire_s(buf_a, (ng - 2) * G, ss_a)
    drain_s(buf_a, ss_a)
    drain_g(buf_b, gs_b)
    fire_s(buf_b, (ng - 1) * G, ss_b)
    drain_s(buf_b, ss_b)

    plsc.subcore_barrier()
    pltpu.sync_copy(acc.at[pl.ds(s * RPT, RPT)],
                    out_hbm.at[c, pl.ds(s * RPT, RPT)])

  return agg


def _make_deg():
  LAG = 8

  @functools.partial(
      pl.kernel,
      out_type=jax.ShapeDtypeStruct((NC, PN, 16), jnp.float32),
      mesh=_mesh(),
      compiler_params=pltpu.CompilerParams(use_tc_tiling_on_sc=False),
      scratch_types=[
          pltpu.VMEM((CMAX, K), jnp.int32),
          pltpu.VMEM((K, 16), jnp.float32),
          pltpu.VMEM((ZR, 16), jnp.float32),
          pltpu.VMEM_SHARED((PN, 16), jnp.float32),
          pltpu.SemaphoreType.DMA,
      ],
  )
  def deg(dst_hbm, out_hbm, didx, obuf, zbuf, acc, sem):
    c = lax.axis_index("c")
    s = lax.axis_index("s")
    start = jnp.where(c == 0, s * C0, NS * C0 + s * C1)
    nch = jnp.where(c == 0, C0, C1)
    z = jnp.zeros((16,), jnp.float32)
    one = jnp.full((16,), 1.0, jnp.float32)
    for i in range(ZR):
      zbuf[i, pl.ds(0, 16)] = z
    for i in range(K):
      obuf[i, pl.ds(0, 16)] = one
    for j in range(RPT // ZR):
      pltpu.sync_copy(zbuf, acc.at[pl.ds(s * RPT + j * ZR, ZR)])
    pltpu.sync_copy(dst_hbm.at[pl.ds(start, CMAX)], didx)
    plsc.subcore_barrier()

    def chunk(i, carry):
      pltpu.async_copy(obuf, acc.at[didx.at[i]], sem, add=True)

      @pl.when(i >= LAG)
      def _():
        pltpu.make_async_copy(obuf, acc.at[didx.at[0]], sem).wait()

      return carry

    lax.fori_loop(0, nch, chunk, 0)
    for _ in range(LAG):
      pltpu.make_async_copy(obuf, acc.at[didx.at[0]], sem).wait()
    plsc.subcore_barrier()
    pltpu.sync_copy(acc.at[pl.ds(s * RPT, RPT)],
                    out_hbm.at[c, pl.ds(s * RPT, RPT)])

  return deg


_deg_kernel = _make_deg()
_agg16 = _make_edge_agg(16)
_agg32 = _make_edge_agg(32)


def _tc1(x, W1, degp):
  def body(x_ref, w_ref, d_ref, h_ref, dinv_ref):
    deg = d_ref[0, pl.ds(0, N), 0:1] + d_ref[1, pl.ds(0, N), 0:1] + 1.0
    dinv = lax.rsqrt(deg)
    h = jnp.dot(x_ref[...], w_ref[...], preferred_element_type=jnp.float32)
    h_ref[pl.ds(0, N), :] = h * dinv
    h_ref[pl.ds(N, PN - N), :] = jnp.zeros((PN - N, 16), jnp.float32)
    dinv_ref[...] = dinv

  return pl.pallas_call(
      body,
      out_shape=(jax.ShapeDtypeStruct((PN, 16), jnp.float32),
                 jax.ShapeDtypeStruct((N, 1), jnp.float32)),
  )(x, W1, degp)


def _tc2(pp, hp, dinv, b1, W2):
  def body(p_ref, hp_ref, dinv_ref, b1_ref, w2_ref, out_ref):
    dinv = dinv_ref[...]
    agg = (p_ref[0, pl.ds(0, N), :] + p_ref[1, pl.ds(0, N), :]
           + hp_ref[pl.ds(0, N), :]) * dinv
    out1 = jnp.maximum(agg + b1_ref[...], 0.0)
    h2 = jnp.dot(out1, w2_ref[...], preferred_element_type=jnp.float32)
    out_ref[pl.ds(0, N), :] = h2 * dinv
    out_ref[pl.ds(N, PN - N), :] = jnp.zeros((PN - N, 32), jnp.float32)

  return pl.pallas_call(
      body,
      out_shape=jax.ShapeDtypeStruct((PN, 32), jnp.float32),
  )(pp, hp, dinv, b1, W2)


def _tc3(pp, hp, dinv, b2, Wf1, bf1, Wf2, bf2, Wf3, bf3):
  def body(p_ref, hp_ref, dinv_ref, b2_ref, wf1_ref, bf1_ref,
           wf2_ref, bf2_ref, wf3_ref, bf3_ref, out_ref):
    agg = (p_ref[0, pl.ds(0, N), :] + p_ref[1, pl.ds(0, N), :]
           + hp_ref[pl.ds(0, N), :]) * dinv_ref[...]
    out2 = jnp.maximum(agg + b2_ref[...], 0.0)
    y = jnp.maximum(
        jnp.dot(out2, wf1_ref[...], preferred_element_type=jnp.float32)
        + bf1_ref[...], 0.0)
    y = jnp.maximum(
        jnp.dot(y, wf2_ref[...], preferred_element_type=jnp.float32)
        + bf2_ref[...], 0.0)
    out_ref[...] = (
        jnp.dot(y, wf3_ref[...], preferred_element_type=jnp.float32)
        + bf3_ref[...])

  return pl.pallas_call(
      body,
      out_shape=jax.ShapeDtypeStruct((N, 40), jnp.float32),
  )(pp, hp, dinv, b2, Wf1, bf1, Wf2, bf2, Wf3, bf3)


def kernel(x, edge_index, W1, b1, W2, b2, Wf1, bf1, Wf2, bf2, Wf3, bf3):
  pad = jnp.full((EP - E,), N, jnp.int32)
  src = jnp.concatenate([edge_index[0].astype(jnp.int32), pad]).reshape(
      TOTCH_PAD, K)
  dst = jnp.concatenate([edge_index[1].astype(jnp.int32), pad]).reshape(
      TOTCH_PAD, K)

  degp = _deg_kernel(dst)
  h1p, dinv = _tc1(x, W1, degp)
  p = _agg16(src, dst, h1p)
  h2p = _tc2(p, h1p, dinv, b1.reshape(1, 16), W2)
  q = _agg32(src, dst, h2p)
  return _tc3(q, h2p, dinv, b2.reshape(1, 32), Wf1,
              bf1.reshape(1, 64), Wf2, bf2.reshape(1, 32), Wf3,
              bf3.reshape(1, 40))


# speedup vs baseline: 1.0188x; 1.0188x over previous
import functools

import jax
import jax.numpy as jnp
from jax import lax
from jax.experimental import pallas as pl
from jax.experimental.pallas import tpu as pltpu
from jax.experimental.pallas import tpu_sc as plsc

N = 10000
E = 320000
NC = 2
NS = 16
PN = 10240
K = 128
G = 5
C0 = 130
C1 = 30
CMAX = max(C0, C1)
TOTCH = NS * (C0 + C1)
TOTCH_PAD = TOTCH + CMAX
EP = TOTCH_PAD * K
RPT = PN // NS
ZR = 128


def _mesh():
  return plsc.VectorSubcoreMesh(
      core_axis_name="c", subcore_axis_name="s",
      num_cores=NC, num_subcores=NS)


def _make_edge_agg(F):
  @functools.partial(
      pl.kernel,
      out_type=jax.ShapeDtypeStruct((NC, PN, F), jnp.float32),
      mesh=_mesh(),
      compiler_params=pltpu.CompilerParams(use_tc_tiling_on_sc=False),
      scratch_types=(
          [pltpu.VMEM((CMAX, K), jnp.int32)] * 2
          + [pltpu.VMEM((K, F), jnp.float32)] * (2 * G)
          + [pltpu.VMEM((ZR, F), jnp.float32)]
          + [pltpu.VMEM_SHARED((PN, F), jnp.float32)]
          + [pltpu.SemaphoreType.DMA] * 4
      ),
  )
  def agg(src_hbm, dst_hbm, hp_hbm, out_hbm, *refs):
    sidx, didx = refs[0], refs[1]
    buf_a = refs[2:2 + G]
    buf_b = refs[2 + G:2 + 2 * G]
    zbuf = refs[2 + 2 * G]
    acc = refs[3 + 2 * G]
    gs_a, gs_b, ss_a, ss_b = refs[4 + 2 * G:8 + 2 * G]

    c = lax.axis_index("c")
    s = lax.axis_index("s")
    start = jnp.where(c == 0, s * C0, NS * C0 + s * C1)
    ng = jnp.where(c == 0, C0 // G, C1 // G)
    np_ = ng // 2

    pltpu.sync_copy(src_hbm.at[pl.ds(start, CMAX)], sidx)
    pltpu.sync_copy(dst_hbm.at[pl.ds(start, CMAX)], didx)
    z = jnp.zeros((16,), jnp.float32)
    for i in range(ZR):
      for j in range(F // 16):
        zbuf[i, pl.ds(16 * j, 16)] = z
    for j in range(RPT // ZR):
      pltpu.sync_copy(zbuf, acc.at[pl.ds(s * RPT + j * ZR, ZR)])

    def fire_g(bufs, base, sem):
      for t in range(G):
        pltpu.async_copy(hp_hbm.at[sidx.at[base + t]], bufs[t], sem)

    def drain_g(bufs, sem):
      for t in range(G):
        pltpu.make_async_copy(hp_hbm.at[sidx.at[0]], bufs[t], sem).wait()

    def fire_s(bufs, base, sem):
      for t in range(G):
        pltpu.async_copy(bufs[t], acc.at[didx.at[base + t]], sem, add=True)

    def drain_s(bufs, sem):
      for t in range(G):
        pltpu.make_async_copy(bufs[t], acc.at[didx.at[0]], sem).wait()

    fire_g(buf_a, 0, gs_a)
    plsc.subcore_barrier()

    def pair(p, carry):
      a0 = (2 * p) * G
      b1 = (2 * p + 1) * G
      a2 = (2 * p + 2) * G
      fire_g(buf_b, b1, gs_b)
      drain_g(buf_a, gs_a)
      fire_s(buf_a, a0, ss_a)
      drain_s(buf_a, ss_a)
      fire_g(buf_a, a2, gs_a)
      drain_g(buf_b, gs_b)
      fire_s(buf_b, b1, ss_b)
      drain_s(buf_b, ss_b)
      return carry

    lax.fori_loop(0, np_ - 1, pair, 0)

    fire_g(buf_b, (ng - 1) * G, gs_b)
    drain_g(buf_a, gs_a)
    fire_s(buf_a, (ng - 2) * G, ss_a)
    drain_s(buf_a, ss_a)
    drain_g(buf_b, gs_b)
    fire_s(buf_b, (ng - 1) * G, ss_b)
    drain_s(buf_b, ss_b)

    plsc.subcore_barrier()
    pltpu.sync_copy(acc.at[pl.ds(s * RPT, RPT)],
                    out_hbm.at[c, pl.ds(s * RPT, RPT)])

  return agg


def _make_deg():
  LAG = 8

  @functools.partial(
      pl.kernel,
      out_type=jax.ShapeDtypeStruct((NC, PN, 16), jnp.float32),
      mesh=_mesh(),
      compiler_params=pltpu.CompilerParams(use_tc_tiling_on_sc=False),
      scratch_types=[
          pltpu.VMEM((CMAX, K), jnp.int32),
          pltpu.VMEM((K, 16), jnp.float32),
          pltpu.VMEM((ZR, 16), jnp.float32),
          pltpu.VMEM_SHARED((PN, 16), jnp.float32),
          pltpu.SemaphoreType.DMA,
      ],
  )
  def deg(dst_hbm, out_hbm, didx, obuf, zbuf, acc, sem):
    c = lax.axis_index("c")
    s = lax.axis_index("s")
    start = jnp.where(c == 0, s * C0, NS * C0 + s * C1)
    nch = jnp.where(c == 0, C0, C1)
    z = jnp.zeros((16,), jnp.float32)
    one = jnp.full((16,), 1.0, jnp.float32)
    for i in range(ZR):
      zbuf[i, pl.ds(0, 16)] = z
    for i in range(K):
      obuf[i, pl.ds(0, 16)] = one
    for j in range(RPT // ZR):
      pltpu.sync_copy(zbuf, acc.at[pl.ds(s * RPT + j * ZR, ZR)])
    pltpu.sync_copy(dst_hbm.at[pl.ds(start, CMAX)], didx)
    plsc.subcore_barrier()

    def chunk(i, carry):
      pltpu.async_copy(obuf, acc.at[didx.at[i]], sem, add=True)

      @pl.when(i >= LAG)
      def _():
        pltpu.make_async_copy(obuf, acc.at[didx.at[0]], sem).wait()

      return carry

    lax.fori_loop(0, nch, chunk, 0)
    for _ in range(LAG):
      pltpu.make_async_copy(obuf, acc.at[didx.at[0]], sem).wait()
    plsc.subcore_barrier()
    pltpu.sync_copy(acc.at[pl.ds(s * RPT, RPT)],
                    out_hbm.at[c, pl.ds(s * RPT, RPT)])

  return deg


_deg_kernel = _make_deg()
_agg16 = _make_edge_agg(16)
_agg32 = _make_edge_agg(32)


def _tc1(x, W1, degp):
  def body(x_ref, w_ref, d_ref, h_ref, dinv_ref):
    deg = d_ref[0, pl.ds(0, N), 0:1] + d_ref[1, pl.ds(0, N), 0:1] + 1.0
    dinv = lax.rsqrt(deg)
    h = jnp.dot(x_ref[...], w_ref[...], preferred_element_type=jnp.float32)
    h_ref[pl.ds(0, N), :] = h * dinv
    h_ref[pl.ds(N, PN - N), :] = jnp.zeros((PN - N, 16), jnp.float32)
    dinv_ref[...] = dinv

  return pl.pallas_call(
      body,
      out_shape=(jax.ShapeDtypeStruct((PN, 16), jnp.float32),
                 jax.ShapeDtypeStruct((N, 1), jnp.float32)),
  )(x, W1, degp)


def _tc2(pp, hp, dinv, b1, W2):
  def body(p_ref, hp_ref, dinv_ref, b1_ref, w2_ref, out_ref):
    dinv = dinv_ref[...]
    agg = (p_ref[0, pl.ds(0, N), :] + p_ref[1, pl.ds(0, N), :]
           + hp_ref[pl.ds(0, N), :]) * dinv
    out1 = jnp.maximum(agg + b1_ref[...], 0.0)
    h2 = jnp.dot(out1, w2_ref[...], preferred_element_type=jnp.float32)
    out_ref[pl.ds(0, N), :] = h2 * dinv
    out_ref[pl.ds(N, PN - N), :] = jnp.zeros((PN - N, 32), jnp.float32)

  return pl.pallas_call(
      body,
      out_shape=jax.ShapeDtypeStruct((PN, 32), jnp.float32),
  )(pp, hp, dinv, b1, W2)


def _tc3(pp, hp, dinv, b2, Wf1, bf1, Wf2, bf2, Wf3, bf3):
  def body(p_ref, hp_ref, dinv_ref, b2_ref, wf1_ref, bf1_ref,
           wf2_ref, bf2_ref, wf3_ref, bf3_ref, out_ref):
    agg = (p_ref[0, pl.ds(0, N), :] + p_ref[1, pl.ds(0, N), :]
           + hp_ref[pl.ds(0, N), :]) * dinv_ref[...]
    out2 = jnp.maximum(agg + b2_ref[...], 0.0)
    y = jnp.maximum(
        jnp.dot(out2, wf1_ref[...], preferred_element_type=jnp.float32)
        + bf1_ref[...], 0.0)
    y = jnp.maximum(
        jnp.dot(y, wf2_ref[...], preferred_element_type=jnp.float32)
        + bf2_ref[...], 0.0)
    out_ref[...] = (
        jnp.dot(y, wf3_ref[...], preferred_element_type=jnp.float32)
        + bf3_ref[...])

  return pl.pallas_call(
      body,
      out_shape=jax.ShapeDtypeStruct((N, 40), jnp.float32),
  )(pp, hp, dinv, b2, Wf1, bf1, Wf2, bf2, Wf3, bf3)


def kernel(x, edge_index, W1, b1, W2, b2, Wf1, bf1, Wf2, bf2, Wf3, bf3):
  pad = jnp.full((EP - E,), N, jnp.int32)
  src = jnp.concatenate([edge_index[0].astype(jnp.int32), pad]).reshape(
      TOTCH_PAD, K)
  dst = jnp.concatenate([edge_index[1].astype(jnp.int32), pad]).reshape(
      TOTCH_PAD, K)

  degp = _deg_kernel(dst)
  h1p, dinv = _tc1(x, W1, degp)
  p = _agg16(src, dst, h1p)
  h2p = _tc2(p, h1p, dinv, b1.reshape(1, 16), W2)
  q = _agg32(src, dst, h2p)
  return _tc3(q, h2p, dinv, b2.reshape(1, 32), Wf1,
              bf1.reshape(1, 64), Wf2, bf2.reshape(1, 32), Wf3,
              bf3.reshape(1, 40))


# speedup vs baseline: 1.0986x; 1.0783x over previous
import functools

import jax
import jax.numpy as jnp
from jax import lax
from jax.experimental import pallas as pl
from jax.experimental.pallas import tpu as pltpu
from jax.experimental.pallas import tpu_sc as plsc

N = 10000
E = 320000
NC = 2
NS = 16
PN = 10240
K = 128
G = 5
C0 = 110
C1 = 50
CMAX = max(C0, C1)
TOTCH = NS * (C0 + C1)
TOTCH_PAD = TOTCH + CMAX
EP = TOTCH_PAD * K
RPT = PN // NS
ZR = 128


def _mesh():
  return plsc.VectorSubcoreMesh(
      core_axis_name="c", subcore_axis_name="s",
      num_cores=NC, num_subcores=NS)


def _make_edge_agg(F):
  @functools.partial(
      pl.kernel,
      out_type=jax.ShapeDtypeStruct((NC, PN, F), jnp.float32),
      mesh=_mesh(),
      compiler_params=pltpu.CompilerParams(use_tc_tiling_on_sc=False),
      scratch_types=(
          [pltpu.VMEM((CMAX, K), jnp.int32)] * 2
          + [pltpu.VMEM((K, F), jnp.float32)] * (2 * G)
          + [pltpu.VMEM((ZR, F), jnp.float32)]
          + [pltpu.VMEM_SHARED((PN, F), jnp.float32)]
          + [pltpu.SemaphoreType.DMA] * 4
      ),
  )
  def agg(src_hbm, dst_hbm, hp_hbm, out_hbm, *refs):
    sidx, didx = refs[0], refs[1]
    buf_a = refs[2:2 + G]
    buf_b = refs[2 + G:2 + 2 * G]
    zbuf = refs[2 + 2 * G]
    acc = refs[3 + 2 * G]
    gs_a, gs_b, ss_a, ss_b = refs[4 + 2 * G:8 + 2 * G]

    c = lax.axis_index("c")
    s = lax.axis_index("s")
    start = jnp.where(c == 0, s * C0, NS * C0 + s * C1)
    ng = jnp.where(c == 0, C0 // G, C1 // G)
    np_ = ng // 2

    pltpu.sync_copy(src_hbm.at[pl.ds(start, CMAX)], sidx)
    pltpu.sync_copy(dst_hbm.at[pl.ds(start, CMAX)], didx)
    z = jnp.zeros((16,), jnp.float32)
    for i in range(ZR):
      for j in range(F // 16):
        zbuf[i, pl.ds(16 * j, 16)] = z
    for j in range(RPT // ZR):
      pltpu.sync_copy(zbuf, acc.at[pl.ds(s * RPT + j * ZR, ZR)])

    def fire_g(bufs, base, sem):
      for t in range(G):
        pltpu.async_copy(hp_hbm.at[sidx.at[base + t]], bufs[t], sem)

    def drain_g(bufs, sem):
      for t in range(G):
        pltpu.make_async_copy(hp_hbm.at[sidx.at[0]], bufs[t], sem).wait()

    def fire_s(bufs, base, sem):
      for t in range(G):
        pltpu.async_copy(bufs[t], acc.at[didx.at[base + t]], sem, add=True)

    def drain_s(bufs, sem):
      for t in range(G):
        pltpu.make_async_copy(bufs[t], acc.at[didx.at[0]], sem).wait()

    fire_g(buf_a, 0, gs_a)
    plsc.subcore_barrier()

    def pair(p, carry):
      a0 = (2 * p) * G
      b1 = (2 * p + 1) * G
      a2 = (2 * p + 2) * G
      fire_g(buf_b, b1, gs_b)
      drain_g(buf_a, gs_a)
      fire_s(buf_a, a0, ss_a)
      drain_s(buf_a, ss_a)
      fire_g(buf_a, a2, gs_a)
      drain_g(buf_b, gs_b)
      fire_s(buf_b, b1, ss_b)
      drain_s(buf_b, ss_b)
      return carry

    lax.fori_loop(0, np_ - 1, pair, 0)

    fire_g(buf_b, (ng - 1) * G, gs_b)
    drain_g(buf_a, gs_a)
    fire_s(buf_a, (ng - 2) * G, ss_a)
    drain_s(buf_a, ss_a)
    drain_g(buf_b, gs_b)
    fire_s(buf_b, (ng - 1) * G, ss_b)
    drain_s(buf_b, ss_b)

    plsc.subcore_barrier()
    pltpu.sync_copy(acc.at[pl.ds(s * RPT, RPT)],
                    out_hbm.at[c, pl.ds(s * RPT, RPT)])

  return agg


def _make_deg():
  LAG = 8

  @functools.partial(
      pl.kernel,
      out_type=jax.ShapeDtypeStruct((NC, PN, 16), jnp.float32),
      mesh=_mesh(),
      compiler_params=pltpu.CompilerParams(use_tc_tiling_on_sc=False),
      scratch_types=[
          pltpu.VMEM((CMAX, K), jnp.int32),
          pltpu.VMEM((K, 16), jnp.float32),
          pltpu.VMEM((ZR, 16), jnp.float32),
          pltpu.VMEM_SHARED((PN, 16), jnp.float32),
          pltpu.SemaphoreType.DMA,
      ],
  )
  def deg(dst_hbm, out_hbm, didx, obuf, zbuf, acc, sem):
    c = lax.axis_index("c")
    s = lax.axis_index("s")
    start = jnp.where(c == 0, s * C0, NS * C0 + s * C1)
    nch = jnp.where(c == 0, C0, C1)
    z = jnp.zeros((16,), jnp.float32)
    one = jnp.full((16,), 1.0, jnp.float32)
    for i in range(ZR):
      zbuf[i, pl.ds(0, 16)] = z
    for i in range(K):
      obuf[i, pl.ds(0, 16)] = one
    for j in range(RPT // ZR):
      pltpu.sync_copy(zbuf, acc.at[pl.ds(s * RPT + j * ZR, ZR)])
    pltpu.sync_copy(dst_hbm.at[pl.ds(start, CMAX)], didx)
    plsc.subcore_barrier()

    def chunk(i, carry):
      pltpu.async_copy(obuf, acc.at[didx.at[i]], sem, add=True)

      @pl.when(i >= LAG)
      def _():
        pltpu.make_async_copy(obuf, acc.at[didx.at[0]], sem).wait()

      return carry

    lax.fori_loop(0, nch, chunk, 0)
    for _ in range(LAG):
      pltpu.make_async_copy(obuf, acc.at[didx.at[0]], sem).wait()
    plsc.subcore_barrier()
    pltpu.sync_copy(acc.at[pl.ds(s * RPT, RPT)],
                    out_hbm.at[c, pl.ds(s * RPT, RPT)])

  return deg


_deg_kernel = _make_deg()
_agg16 = _make_edge_agg(16)
_agg32 = _make_edge_agg(32)


def _tc1(x, W1, degp):
  def body(x_ref, w_ref, d_ref, h_ref, dinv_ref):
    deg = d_ref[0, pl.ds(0, N), 0:1] + d_ref[1, pl.ds(0, N), 0:1] + 1.0
    dinv = lax.rsqrt(deg)
    h = jnp.dot(x_ref[...], w_ref[...], preferred_element_type=jnp.float32)
    h_ref[pl.ds(0, N), :] = h * dinv
    h_ref[pl.ds(N, PN - N), :] = jnp.zeros((PN - N, 16), jnp.float32)
    dinv_ref[...] = dinv

  return pl.pallas_call(
      body,
      out_shape=(jax.ShapeDtypeStruct((PN, 16), jnp.float32),
                 jax.ShapeDtypeStruct((N, 1), jnp.float32)),
  )(x, W1, degp)


def _tc2(pp, hp, dinv, b1, W2):
  def body(p_ref, hp_ref, dinv_ref, b1_ref, w2_ref, out_ref):
    dinv = dinv_ref[...]
    agg = (p_ref[0, pl.ds(0, N), :] + p_ref[1, pl.ds(0, N), :]
           + hp_ref[pl.ds(0, N), :]) * dinv
    out1 = jnp.maximum(agg + b1_ref[...], 0.0)
    h2 = jnp.dot(out1, w2_ref[...], preferred_element_type=jnp.float32)
    out_ref[pl.ds(0, N), :] = h2 * dinv
    out_ref[pl.ds(N, PN - N), :] = jnp.zeros((PN - N, 32), jnp.float32)

  return pl.pallas_call(
      body,
      out_shape=jax.ShapeDtypeStruct((PN, 32), jnp.float32),
  )(pp, hp, dinv, b1, W2)


def _tc3(pp, hp, dinv, b2, Wf1, bf1, Wf2, bf2, Wf3, bf3):
  def body(p_ref, hp_ref, dinv_ref, b2_ref, wf1_ref, bf1_ref,
           wf2_ref, bf2_ref, wf3_ref, bf3_ref, out_ref):
    agg = (p_ref[0, pl.ds(0, N), :] + p_ref[1, pl.ds(0, N), :]
           + hp_ref[pl.ds(0, N), :]) * dinv_ref[...]
    out2 = jnp.maximum(agg + b2_ref[...], 0.0)
    y = jnp.maximum(
        jnp.dot(out2, wf1_ref[...], preferred_element_type=jnp.float32)
        + bf1_ref[...], 0.0)
    y = jnp.maximum(
        jnp.dot(y, wf2_ref[...], preferred_element_type=jnp.float32)
        + bf2_ref[...], 0.0)
    out_ref[...] = (
        jnp.dot(y, wf3_ref[...], preferred_element_type=jnp.float32)
        + bf3_ref[...])

  return pl.pallas_call(
      body,
      out_shape=jax.ShapeDtypeStruct((N, 40), jnp.float32),
  )(pp, hp, dinv, b2, Wf1, bf1, Wf2, bf2, Wf3, bf3)


def kernel(x, edge_index, W1, b1, W2, b2, Wf1, bf1, Wf2, bf2, Wf3, bf3):
  pad = jnp.full((EP - E,), N, jnp.int32)
  src = jnp.concatenate([edge_index[0].astype(jnp.int32), pad]).reshape(
      TOTCH_PAD, K)
  dst = jnp.concatenate([edge_index[1].astype(jnp.int32), pad]).reshape(
      TOTCH_PAD, K)

  degp = _deg_kernel(dst)
  h1p, dinv = _tc1(x, W1, degp)
  p = _agg16(src, dst, h1p)
  h2p = _tc2(p, h1p, dinv, b1.reshape(1, 16), W2)
  q = _agg32(src, dst, h2p)
  return _tc3(q, h2p, dinv, b2.reshape(1, 32), Wf1,
              bf1.reshape(1, 64), Wf2, bf2.reshape(1, 32), Wf3,
              bf3.reshape(1, 40))


# speedup vs baseline: 1.1059x; 1.0067x over previous
import functools

import jax
import jax.numpy as jnp
from jax import lax
from jax.experimental import pallas as pl
from jax.experimental.pallas import tpu as pltpu
from jax.experimental.pallas import tpu_sc as plsc

N = 10000
E = 320000
NC = 2
NS = 16
PN = 10240
K = 128
G = 5
C0 = 120
C1 = 40
CMAX = max(C0, C1)
TOTCH = NS * (C0 + C1)
TOTCH_PAD = TOTCH + CMAX
EP = TOTCH_PAD * K
RPT = PN // NS
ZR = 128


def _mesh():
  return plsc.VectorSubcoreMesh(
      core_axis_name="c", subcore_axis_name="s",
      num_cores=NC, num_subcores=NS)


def _make_edge_agg(F):
  @functools.partial(
      pl.kernel,
      out_type=jax.ShapeDtypeStruct((NC, PN, F), jnp.float32),
      mesh=_mesh(),
      compiler_params=pltpu.CompilerParams(use_tc_tiling_on_sc=False),
      scratch_types=(
          [pltpu.VMEM((CMAX, K), jnp.int32)] * 2
          + [pltpu.VMEM((K, F), jnp.float32)] * (2 * G)
          + [pltpu.VMEM((ZR, F), jnp.float32)]
          + [pltpu.VMEM_SHARED((PN, F), jnp.float32)]
          + [pltpu.SemaphoreType.DMA] * 4
      ),
  )
  def agg(src_hbm, dst_hbm, hp_hbm, out_hbm, *refs):
    sidx, didx = refs[0], refs[1]
    buf_a = refs[2:2 + G]
    buf_b = refs[2 + G:2 + 2 * G]
    zbuf = refs[2 + 2 * G]
    acc = refs[3 + 2 * G]
    gs_a, gs_b, ss_a, ss_b = refs[4 + 2 * G:8 + 2 * G]

    c = lax.axis_index("c")
    s = lax.axis_index("s")
    start = jnp.where(c == 0, s * C0, NS * C0 + s * C1)
    ng = jnp.where(c == 0, C0 // G, C1 // G)
    np_ = ng // 2

    pltpu.sync_copy(src_hbm.at[pl.ds(start, CMAX)], sidx)
    pltpu.sync_copy(dst_hbm.at[pl.ds(start, CMAX)], didx)
    z = jnp.zeros((16,), jnp.float32)
    for i in range(ZR):
      for j in range(F // 16):
        zbuf[i, pl.ds(16 * j, 16)] = z
    for j in range(RPT // ZR):
      pltpu.sync_copy(zbuf, acc.at[pl.ds(s * RPT + j * ZR, ZR)])

    def fire_g(bufs, base, sem):
      for t in range(G):
        pltpu.async_copy(hp_hbm.at[sidx.at[base + t]], bufs[t], sem)

    def drain_g(bufs, sem):
      for t in range(G):
        pltpu.make_async_copy(hp_hbm.at[sidx.at[0]], bufs[t], sem).wait()

    def fire_s(bufs, base, sem):
      for t in range(G):
        pltpu.async_copy(bufs[t], acc.at[didx.at[base + t]], sem, add=True)

    def drain_s(bufs, sem):
      for t in range(G):
        pltpu.make_async_copy(bufs[t], acc.at[didx.at[0]], sem).wait()

    fire_g(buf_a, 0, gs_a)
    plsc.subcore_barrier()

    def pair(p, carry):
      a0 = (2 * p) * G
      b1 = (2 * p + 1) * G
      a2 = (2 * p + 2) * G
      fire_g(buf_b, b1, gs_b)
      drain_g(buf_a, gs_a)
      fire_s(buf_a, a0, ss_a)
      drain_s(buf_a, ss_a)
      fire_g(buf_a, a2, gs_a)
      drain_g(buf_b, gs_b)
      fire_s(buf_b, b1, ss_b)
      drain_s(buf_b, ss_b)
      return carry

    lax.fori_loop(0, np_ - 1, pair, 0)

    fire_g(buf_b, (ng - 1) * G, gs_b)
    drain_g(buf_a, gs_a)
    fire_s(buf_a, (ng - 2) * G, ss_a)
    drain_s(buf_a, ss_a)
    drain_g(buf_b, gs_b)
    fire_s(buf_b, (ng - 1) * G, ss_b)
    drain_s(buf_b, ss_b)

    plsc.subcore_barrier()
    pltpu.sync_copy(acc.at[pl.ds(s * RPT, RPT)],
                    out_hbm.at[c, pl.ds(s * RPT, RPT)])

  return agg


def _make_deg():
  LAG = 8

  @functools.partial(
      pl.kernel,
      out_type=jax.ShapeDtypeStruct((NC, PN, 16), jnp.float32),
      mesh=_mesh(),
      compiler_params=pltpu.CompilerParams(use_tc_tiling_on_sc=False),
      scratch_types=[
          pltpu.VMEM((CMAX, K), jnp.int32),
          pltpu.VMEM((K, 16), jnp.float32),
          pltpu.VMEM((ZR, 16), jnp.float32),
          pltpu.VMEM_SHARED((PN, 16), jnp.float32),
          pltpu.SemaphoreType.DMA,
      ],
  )
  def deg(dst_hbm, out_hbm, didx, obuf, zbuf, acc, sem):
    c = lax.axis_index("c")
    s = lax.axis_index("s")
    start = jnp.where(c == 0, s * C0, NS * C0 + s * C1)
    nch = jnp.where(c == 0, C0, C1)
    z = jnp.zeros((16,), jnp.float32)
    one = jnp.full((16,), 1.0, jnp.float32)
    for i in range(ZR):
      zbuf[i, pl.ds(0, 16)] = z
    for i in range(K):
      obuf[i, pl.ds(0, 16)] = one
    for j in range(RPT // ZR):
      pltpu.sync_copy(zbuf, acc.at[pl.ds(s * RPT + j * ZR, ZR)])
    pltpu.sync_copy(dst_hbm.at[pl.ds(start, CMAX)], didx)
    plsc.subcore_barrier()

    def chunk(i, carry):
      pltpu.async_copy(obuf, acc.at[didx.at[i]], sem, add=True)

      @pl.when(i >= LAG)
      def _():
        pltpu.make_async_copy(obuf, acc.at[didx.at[0]], sem).wait()

      return carry

    lax.fori_loop(0, nch, chunk, 0)
    for _ in range(LAG):
      pltpu.make_async_copy(obuf, acc.at[didx.at[0]], sem).wait()
    plsc.subcore_barrier()
    pltpu.sync_copy(acc.at[pl.ds(s * RPT, RPT)],
                    out_hbm.at[c, pl.ds(s * RPT, RPT)])

  return deg


_deg_kernel = _make_deg()
_agg16 = _make_edge_agg(16)
_agg32 = _make_edge_agg(32)


def _tc1(x, W1, degp):
  def body(x_ref, w_ref, d_ref, h_ref, dinv_ref):
    deg = d_ref[0, pl.ds(0, N), 0:1] + d_ref[1, pl.ds(0, N), 0:1] + 1.0
    dinv = lax.rsqrt(deg)
    h = jnp.dot(x_ref[...], w_ref[...], preferred_element_type=jnp.float32)
    h_ref[pl.ds(0, N), :] = h * dinv
    h_ref[pl.ds(N, PN - N), :] = jnp.zeros((PN - N, 16), jnp.float32)
    dinv_ref[...] = dinv

  return pl.pallas_call(
      body,
      out_shape=(jax.ShapeDtypeStruct((PN, 16), jnp.float32),
                 jax.ShapeDtypeStruct((N, 1), jnp.float32)),
  )(x, W1, degp)


def _tc2(pp, hp, dinv, b1, W2):
  def body(p_ref, hp_ref, dinv_ref, b1_ref, w2_ref, out_ref):
    dinv = dinv_ref[...]
    agg = (p_ref[0, pl.ds(0, N), :] + p_ref[1, pl.ds(0, N), :]
           + hp_ref[pl.ds(0, N), :]) * dinv
    out1 = jnp.maximum(agg + b1_ref[...], 0.0)
    h2 = jnp.dot(out1, w2_ref[...], preferred_element_type=jnp.float32)
    out_ref[pl.ds(0, N), :] = h2 * dinv
    out_ref[pl.ds(N, PN - N), :] = jnp.zeros((PN - N, 32), jnp.float32)

  return pl.pallas_call(
      body,
      out_shape=jax.ShapeDtypeStruct((PN, 32), jnp.float32),
  )(pp, hp, dinv, b1, W2)


def _tc3(pp, hp, dinv, b2, Wf1, bf1, Wf2, bf2, Wf3, bf3):
  def body(p_ref, hp_ref, dinv_ref, b2_ref, wf1_ref, bf1_ref,
           wf2_ref, bf2_ref, wf3_ref, bf3_ref, out_ref):
    agg = (p_ref[0, pl.ds(0, N), :] + p_ref[1, pl.ds(0, N), :]
           + hp_ref[pl.ds(0, N), :]) * dinv_ref[...]
    out2 = jnp.maximum(agg + b2_ref[...], 0.0)
    y = jnp.maximum(
        jnp.dot(out2, wf1_ref[...], preferred_element_type=jnp.float32)
        + bf1_ref[...], 0.0)
    y = jnp.maximum(
        jnp.dot(y, wf2_ref[...], preferred_element_type=jnp.float32)
        + bf2_ref[...], 0.0)
    out_ref[...] = (
        jnp.dot(y, wf3_ref[...], preferred_element_type=jnp.float32)
        + bf3_ref[...])

  return pl.pallas_call(
      body,
      out_shape=jax.ShapeDtypeStruct((N, 40), jnp.float32),
  )(pp, hp, dinv, b2, Wf1, bf1, Wf2, bf2, Wf3, bf3)


def kernel(x, edge_index, W1, b1, W2, b2, Wf1, bf1, Wf2, bf2, Wf3, bf3):
  pad = jnp.full((EP - E,), N, jnp.int32)
  src = jnp.concatenate([edge_index[0].astype(jnp.int32), pad]).reshape(
      TOTCH_PAD, K)
  dst = jnp.concatenate([edge_index[1].astype(jnp.int32), pad]).reshape(
      TOTCH_PAD, K)

  degp = _deg_kernel(dst)
  h1p, dinv = _tc1(x, W1, degp)
  p = _agg16(src, dst, h1p)
  h2p = _tc2(p, h1p, dinv, b1.reshape(1, 16), W2)
  q = _agg32(src, dst, h2p)
  return _tc3(q, h2p, dinv, b2.reshape(1, 32), Wf1,
              bf1.reshape(1, 64), Wf2, bf2.reshape(1, 32), Wf3,
              bf3.reshape(1, 40))


# speedup vs baseline: 1.6626x; 1.5034x over previous
import functools

import jax
import jax.numpy as jnp
from jax import lax
from jax.experimental import pallas as pl
from jax.experimental.pallas import tpu as pltpu
from jax.experimental.pallas import tpu_sc as plsc

N = 10000
E = 320000
NC = 2
NS = 16
PN = 10240
K = 128
G = 5
C0 = 80
C1 = 80
CMAX = max(C0, C1)
TOTCH = NS * (C0 + C1)
TOTCH_PAD = TOTCH + CMAX
EP = TOTCH_PAD * K
RPT = PN // NS
ZR = 128


def _mesh():
  return plsc.VectorSubcoreMesh(
      core_axis_name="c", subcore_axis_name="s",
      num_cores=NC, num_subcores=NS)


def _make_edge_agg(F):
  @functools.partial(
      pl.kernel,
      out_type=jax.ShapeDtypeStruct((NC, PN, F), jnp.float32),
      mesh=_mesh(),
      compiler_params=pltpu.CompilerParams(use_tc_tiling_on_sc=False),
      scratch_types=(
          [pltpu.VMEM((CMAX, K), jnp.int32)] * 2
          + [pltpu.VMEM((K, F), jnp.float32)] * (2 * G)
          + [pltpu.VMEM((ZR, F), jnp.float32)]
          + [pltpu.VMEM_SHARED((PN, F), jnp.float32)]
          + [pltpu.VMEM_SHARED((PN, F), jnp.float32)]
          + [pltpu.SemaphoreType.DMA] * 4
      ),
  )
  def agg(src_hbm, dst_hbm, hp_hbm, out_hbm, *refs):
    sidx, didx = refs[0], refs[1]
    buf_a = refs[2:2 + G]
    buf_b = refs[2 + G:2 + 2 * G]
    zbuf = refs[2 + 2 * G]
    acc = refs[3 + 2 * G]
    hp_s = refs[4 + 2 * G]
    gs_a, gs_b, ss_a, ss_b = refs[5 + 2 * G:9 + 2 * G]

    c = lax.axis_index("c")
    s = lax.axis_index("s")
    start = jnp.where(c == 0, s * C0, NS * C0 + s * C1)
    ng = jnp.where(c == 0, C0 // G, C1 // G)
    np_ = ng // 2

    pltpu.sync_copy(src_hbm.at[pl.ds(start, CMAX)], sidx)
    pltpu.sync_copy(dst_hbm.at[pl.ds(start, CMAX)], didx)
    z = jnp.zeros((16,), jnp.float32)
    for i in range(ZR):
      for j in range(F // 16):
        zbuf[i, pl.ds(16 * j, 16)] = z
    for j in range(RPT // ZR):
      pltpu.sync_copy(zbuf, acc.at[pl.ds(s * RPT + j * ZR, ZR)])
    pltpu.sync_copy(hp_hbm.at[pl.ds(s * RPT, RPT)],
                    hp_s.at[pl.ds(s * RPT, RPT)])

    def fire_g(bufs, base, sem):
      for t in range(G):
        pltpu.async_copy(hp_s.at[sidx.at[base + t]], bufs[t], sem)

    def drain_g(bufs, sem):
      for t in range(G):
        pltpu.make_async_copy(hp_s.at[sidx.at[0]], bufs[t], sem).wait()

    def fire_s(bufs, base, sem):
      for t in range(G):
        pltpu.async_copy(bufs[t], acc.at[didx.at[base + t]], sem, add=True)

    def drain_s(bufs, sem):
      for t in range(G):
        pltpu.make_async_copy(bufs[t], acc.at[didx.at[0]], sem).wait()

    fire_g(buf_a, 0, gs_a)
    plsc.subcore_barrier()

    def pair(p, carry):
      a0 = (2 * p) * G
      b1 = (2 * p + 1) * G
      a2 = (2 * p + 2) * G
      fire_g(buf_b, b1, gs_b)
      drain_g(buf_a, gs_a)
      fire_s(buf_a, a0, ss_a)
      drain_s(buf_a, ss_a)
      fire_g(buf_a, a2, gs_a)
      drain_g(buf_b, gs_b)
      fire_s(buf_b, b1, ss_b)
      drain_s(buf_b, ss_b)
      return carry

    lax.fori_loop(0, np_ - 1, pair, 0)

    fire_g(buf_b, (ng - 1) * G, gs_b)
    drain_g(buf_a, gs_a)
    fire_s(buf_a, (ng - 2) * G, ss_a)
    drain_s(buf_a, ss_a)
    drain_g(buf_b, gs_b)
    fire_s(buf_b, (ng - 1) * G, ss_b)
    drain_s(buf_b, ss_b)

    plsc.subcore_barrier()
    pltpu.sync_copy(acc.at[pl.ds(s * RPT, RPT)],
                    out_hbm.at[c, pl.ds(s * RPT, RPT)])

  return agg


def _make_deg():
  LAG = 8

  @functools.partial(
      pl.kernel,
      out_type=jax.ShapeDtypeStruct((NC, PN, 16), jnp.float32),
      mesh=_mesh(),
      compiler_params=pltpu.CompilerParams(use_tc_tiling_on_sc=False),
      scratch_types=[
          pltpu.VMEM((CMAX, K), jnp.int32),
          pltpu.VMEM((K, 16), jnp.float32),
          pltpu.VMEM((ZR, 16), jnp.float32),
          pltpu.VMEM_SHARED((PN, 16), jnp.float32),
          pltpu.SemaphoreType.DMA,
      ],
  )
  def deg(dst_hbm, out_hbm, didx, obuf, zbuf, acc, sem):
    c = lax.axis_index("c")
    s = lax.axis_index("s")
    start = jnp.where(c == 0, s * C0, NS * C0 + s * C1)
    nch = jnp.where(c == 0, C0, C1)
    z = jnp.zeros((16,), jnp.float32)
    one = jnp.full((16,), 1.0, jnp.float32)
    for i in range(ZR):
      zbuf[i, pl.ds(0, 16)] = z
    for i in range(K):
      obuf[i, pl.ds(0, 16)] = one
    for j in range(RPT // ZR):
      pltpu.sync_copy(zbuf, acc.at[pl.ds(s * RPT + j * ZR, ZR)])
    pltpu.sync_copy(dst_hbm.at[pl.ds(start, CMAX)], didx)
    plsc.subcore_barrier()

    def chunk(i, carry):
      pltpu.async_copy(obuf, acc.at[didx.at[i]], sem, add=True)

      @pl.when(i >= LAG)
      def _():
        pltpu.make_async_copy(obuf, acc.at[didx.at[0]], sem).wait()

      return carry

    lax.fori_loop(0, nch, chunk, 0)
    for _ in range(LAG):
      pltpu.make_async_copy(obuf, acc.at[didx.at[0]], sem).wait()
    plsc.subcore_barrier()
    pltpu.sync_copy(acc.at[pl.ds(s * RPT, RPT)],
                    out_hbm.at[c, pl.ds(s * RPT, RPT)])

  return deg


_deg_kernel = _make_deg()
_agg16 = _make_edge_agg(16)
_agg32 = _make_edge_agg(32)


def _tc1(x, W1, degp):
  def body(x_ref, w_ref, d_ref, h_ref, dinv_ref):
    deg = d_ref[0, pl.ds(0, N), 0:1] + d_ref[1, pl.ds(0, N), 0:1] + 1.0
    dinv = lax.rsqrt(deg)
    h = jnp.dot(x_ref[...], w_ref[...], preferred_element_type=jnp.float32)
    h_ref[pl.ds(0, N), :] = h * dinv
    h_ref[pl.ds(N, PN - N), :] = jnp.zeros((PN - N, 16), jnp.float32)
    dinv_ref[...] = dinv

  return pl.pallas_call(
      body,
      out_shape=(jax.ShapeDtypeStruct((PN, 16), jnp.float32),
                 jax.ShapeDtypeStruct((N, 1), jnp.float32)),
  )(x, W1, degp)


def _tc2(pp, hp, dinv, b1, W2):
  def body(p_ref, hp_ref, dinv_ref, b1_ref, w2_ref, out_ref):
    dinv = dinv_ref[...]
    agg = (p_ref[0, pl.ds(0, N), :] + p_ref[1, pl.ds(0, N), :]
           + hp_ref[pl.ds(0, N), :]) * dinv
    out1 = jnp.maximum(agg + b1_ref[...], 0.0)
    h2 = jnp.dot(out1, w2_ref[...], preferred_element_type=jnp.float32)
    out_ref[pl.ds(0, N), :] = h2 * dinv
    out_ref[pl.ds(N, PN - N), :] = jnp.zeros((PN - N, 32), jnp.float32)

  return pl.pallas_call(
      body,
      out_shape=jax.ShapeDtypeStruct((PN, 32), jnp.float32),
  )(pp, hp, dinv, b1, W2)


def _tc3(pp, hp, dinv, b2, Wf1, bf1, Wf2, bf2, Wf3, bf3):
  def body(p_ref, hp_ref, dinv_ref, b2_ref, wf1_ref, bf1_ref,
           wf2_ref, bf2_ref, wf3_ref, bf3_ref, out_ref):
    agg = (p_ref[0, pl.ds(0, N), :] + p_ref[1, pl.ds(0, N), :]
           + hp_ref[pl.ds(0, N), :]) * dinv_ref[...]
    out2 = jnp.maximum(agg + b2_ref[...], 0.0)
    y = jnp.maximum(
        jnp.dot(out2, wf1_ref[...], preferred_element_type=jnp.float32)
        + bf1_ref[...], 0.0)
    y = jnp.maximum(
        jnp.dot(y, wf2_ref[...], preferred_element_type=jnp.float32)
        + bf2_ref[...], 0.0)
    out_ref[...] = (
        jnp.dot(y, wf3_ref[...], preferred_element_type=jnp.float32)
        + bf3_ref[...])

  return pl.pallas_call(
      body,
      out_shape=jax.ShapeDtypeStruct((N, 40), jnp.float32),
  )(pp, hp, dinv, b2, Wf1, bf1, Wf2, bf2, Wf3, bf3)


def kernel(x, edge_index, W1, b1, W2, b2, Wf1, bf1, Wf2, bf2, Wf3, bf3):
  pad = jnp.full((EP - E,), N, jnp.int32)
  src = jnp.concatenate([edge_index[0].astype(jnp.int32), pad]).reshape(
      TOTCH_PAD, K)
  dst = jnp.concatenate([edge_index[1].astype(jnp.int32), pad]).reshape(
      TOTCH_PAD, K)

  degp = _deg_kernel(dst)
  h1p, dinv = _tc1(x, W1, degp)
  p = _agg16(src, dst, h1p)
  h2p = _tc2(p, h1p, dinv, b1.reshape(1, 16), W2)
  q = _agg32(src, dst, h2p)
  return _tc3(q, h2p, dinv, b2.reshape(1, 32), Wf1,
              bf1.reshape(1, 64), Wf2, bf2.reshape(1, 32), Wf3,
              bf3.reshape(1, 40))


# speedup vs baseline: 1.6874x; 1.0149x over previous
import functools

import jax
import jax.numpy as jnp
from jax import lax
from jax.experimental import pallas as pl
from jax.experimental.pallas import tpu as pltpu
from jax.experimental.pallas import tpu_sc as plsc

N = 10000
E = 320000
NC = 2
NS = 16
PN = 10240
K = 128
G = 5
C0 = 80
C1 = 80
CMAX = max(C0, C1)
TOTCH = NS * (C0 + C1)
TOTCH_PAD = TOTCH + CMAX
EP = TOTCH_PAD * K
RPT = PN // NS
ZR = 128


def _mesh():
  return plsc.VectorSubcoreMesh(
      core_axis_name="c", subcore_axis_name="s",
      num_cores=NC, num_subcores=NS)


def _make_edge_agg(F):
  @functools.partial(
      pl.kernel,
      out_type=jax.ShapeDtypeStruct((NC, PN, F), jnp.float32),
      mesh=_mesh(),
      compiler_params=pltpu.CompilerParams(use_tc_tiling_on_sc=False),
      scratch_types=(
          [pltpu.VMEM((CMAX, K), jnp.int32)] * 2
          + [pltpu.VMEM((K, F), jnp.float32)] * (2 * G)
          + [pltpu.VMEM((ZR, F), jnp.float32)]
          + [pltpu.VMEM_SHARED((PN, F), jnp.float32)]
          + [pltpu.VMEM_SHARED((PN, F), jnp.float32)]
          + [pltpu.SemaphoreType.DMA] * 4
      ),
  )
  def agg(src_hbm, dst_hbm, hp_hbm, out_hbm, *refs):
    sidx, didx = refs[0], refs[1]
    buf_a = refs[2:2 + G]
    buf_b = refs[2 + G:2 + 2 * G]
    zbuf = refs[2 + 2 * G]
    acc = refs[3 + 2 * G]
    hp_s = refs[4 + 2 * G]
    gs_a, gs_b, ss_a, ss_b = refs[5 + 2 * G:9 + 2 * G]

    c = lax.axis_index("c")
    s = lax.axis_index("s")
    start = jnp.where(c == 0, s * C0, NS * C0 + s * C1)
    ng = jnp.where(c == 0, C0 // G, C1 // G)
    np_ = ng // 2

    pltpu.sync_copy(src_hbm.at[pl.ds(start, CMAX)], sidx)
    pltpu.sync_copy(dst_hbm.at[pl.ds(start, CMAX)], didx)
    z = jnp.zeros((16,), jnp.float32)
    for i in range(ZR):
      for j in range(F // 16):
        zbuf[i, pl.ds(16 * j, 16)] = z
    for j in range(RPT // ZR):
      pltpu.sync_copy(zbuf, acc.at[pl.ds(s * RPT + j * ZR, ZR)])
    pltpu.sync_copy(hp_hbm.at[pl.ds(s * RPT, RPT)],
                    hp_s.at[pl.ds(s * RPT, RPT)])

    def fire_g(bufs, base, sem):
      for t in range(G):
        pltpu.async_copy(hp_s.at[sidx.at[base + t]], bufs[t], sem)

    def drain_g(bufs, sem):
      for t in range(G):
        pltpu.make_async_copy(hp_s.at[sidx.at[0]], bufs[t], sem).wait()

    def fire_s(bufs, base, sem):
      for t in range(G):
        pltpu.async_copy(bufs[t], acc.at[didx.at[base + t]], sem, add=True)

    def drain_s(bufs, sem):
      for t in range(G):
        pltpu.make_async_copy(bufs[t], acc.at[didx.at[0]], sem).wait()

    plsc.subcore_barrier()
    fire_g(buf_a, 0, gs_a)

    def pair(p, carry):
      a0 = (2 * p) * G
      b1 = (2 * p + 1) * G
      a2 = (2 * p + 2) * G
      fire_g(buf_b, b1, gs_b)
      drain_g(buf_a, gs_a)
      fire_s(buf_a, a0, ss_a)
      drain_s(buf_a, ss_a)
      fire_g(buf_a, a2, gs_a)
      drain_g(buf_b, gs_b)
      fire_s(buf_b, b1, ss_b)
      drain_s(buf_b, ss_b)
      return carry

    lax.fori_loop(0, np_ - 1, pair, 0)

    fire_g(buf_b, (ng - 1) * G, gs_b)
    drain_g(buf_a, gs_a)
    fire_s(buf_a, (ng - 2) * G, ss_a)
    drain_s(buf_a, ss_a)
    drain_g(buf_b, gs_b)
    fire_s(buf_b, (ng - 1) * G, ss_b)
    drain_s(buf_b, ss_b)

    plsc.subcore_barrier()
    pltpu.sync_copy(acc.at[pl.ds(s * RPT, RPT)],
                    out_hbm.at[c, pl.ds(s * RPT, RPT)])

  return agg


def _make_deg():
  LAG = 8

  @functools.partial(
      pl.kernel,
      out_type=jax.ShapeDtypeStruct((NC, PN, 16), jnp.float32),
      mesh=_mesh(),
      compiler_params=pltpu.CompilerParams(use_tc_tiling_on_sc=False),
      scratch_types=[
          pltpu.VMEM((CMAX, K), jnp.int32),
          pltpu.VMEM((K, 16), jnp.float32),
          pltpu.VMEM((ZR, 16), jnp.float32),
          pltpu.VMEM_SHARED((PN, 16), jnp.float32),
          pltpu.SemaphoreType.DMA,
      ],
  )
  def deg(dst_hbm, out_hbm, didx, obuf, zbuf, acc, sem):
    c = lax.axis_index("c")
    s = lax.axis_index("s")
    start = jnp.where(c == 0, s * C0, NS * C0 + s * C1)
    nch = jnp.where(c == 0, C0, C1)
    z = jnp.zeros((16,), jnp.float32)
    one = jnp.full((16,), 1.0, jnp.float32)
    for i in range(ZR):
      zbuf[i, pl.ds(0, 16)] = z
    for i in range(K):
      obuf[i, pl.ds(0, 16)] = one
    for j in range(RPT // ZR):
      pltpu.sync_copy(zbuf, acc.at[pl.ds(s * RPT + j * ZR, ZR)])
    pltpu.sync_copy(dst_hbm.at[pl.ds(start, CMAX)], didx)
    plsc.subcore_barrier()

    def chunk(i, carry):
      pltpu.async_copy(obuf, acc.at[didx.at[i]], sem, add=True)

      @pl.when(i >= LAG)
      def _():
        pltpu.make_async_copy(obuf, acc.at[didx.at[0]], sem).wait()

      return carry

    lax.fori_loop(0, nch, chunk, 0)
    for _ in range(LAG):
      pltpu.make_async_copy(obuf, acc.at[didx.at[0]], sem).wait()
    plsc.subcore_barrier()
    pltpu.sync_copy(acc.at[pl.ds(s * RPT, RPT)],
                    out_hbm.at[c, pl.ds(s * RPT, RPT)])

  return deg


_deg_kernel = _make_deg()
_agg16 = _make_edge_agg(16)
_agg32 = _make_edge_agg(32)


def _tc1(x, W1, degp):
  def body(x_ref, w_ref, d_ref, h_ref, dinv_ref):
    deg = d_ref[0, pl.ds(0, N), 0:1] + d_ref[1, pl.ds(0, N), 0:1] + 1.0
    dinv = lax.rsqrt(deg)
    h = jnp.dot(x_ref[...], w_ref[...], preferred_element_type=jnp.float32)
    h_ref[pl.ds(0, N), :] = h * dinv
    h_ref[pl.ds(N, PN - N), :] = jnp.zeros((PN - N, 16), jnp.float32)
    dinv_ref[...] = dinv

  return pl.pallas_call(
      body,
      out_shape=(jax.ShapeDtypeStruct((PN, 16), jnp.float32),
                 jax.ShapeDtypeStruct((N, 1), jnp.float32)),
  )(x, W1, degp)


def _tc2(pp, hp, dinv, b1, W2):
  def body(p_ref, hp_ref, dinv_ref, b1_ref, w2_ref, out_ref):
    dinv = dinv_ref[...]
    agg = (p_ref[0, pl.ds(0, N), :] + p_ref[1, pl.ds(0, N), :]
           + hp_ref[pl.ds(0, N), :]) * dinv
    out1 = jnp.maximum(agg + b1_ref[...], 0.0)
    h2 = jnp.dot(out1, w2_ref[...], preferred_element_type=jnp.float32)
    out_ref[pl.ds(0, N), :] = h2 * dinv
    out_ref[pl.ds(N, PN - N), :] = jnp.zeros((PN - N, 32), jnp.float32)

  return pl.pallas_call(
      body,
      out_shape=jax.ShapeDtypeStruct((PN, 32), jnp.float32),
  )(pp, hp, dinv, b1, W2)


def _tc3(pp, hp, dinv, b2, Wf1, bf1, Wf2, bf2, Wf3, bf3):
  def body(p_ref, hp_ref, dinv_ref, b2_ref, wf1_ref, bf1_ref,
           wf2_ref, bf2_ref, wf3_ref, bf3_ref, out_ref):
    agg = (p_ref[0, pl.ds(0, N), :] + p_ref[1, pl.ds(0, N), :]
           + hp_ref[pl.ds(0, N), :]) * dinv_ref[...]
    out2 = jnp.maximum(agg + b2_ref[...], 0.0)
    y = jnp.maximum(
        jnp.dot(out2, wf1_ref[...], preferred_element_type=jnp.float32)
        + bf1_ref[...], 0.0)
    y = jnp.maximum(
        jnp.dot(y, wf2_ref[...], preferred_element_type=jnp.float32)
        + bf2_ref[...], 0.0)
    out_ref[...] = (
        jnp.dot(y, wf3_ref[...], preferred_element_type=jnp.float32)
        + bf3_ref[...])

  return pl.pallas_call(
      body,
      out_shape=jax.ShapeDtypeStruct((N, 40), jnp.float32),
  )(pp, hp, dinv, b2, Wf1, bf1, Wf2, bf2, Wf3, bf3)


def kernel(x, edge_index, W1, b1, W2, b2, Wf1, bf1, Wf2, bf2, Wf3, bf3):
  pad = jnp.full((EP - E,), N, jnp.int32)
  src = jnp.concatenate([edge_index[0].astype(jnp.int32), pad]).reshape(
      TOTCH_PAD, K)
  dst = jnp.concatenate([edge_index[1].astype(jnp.int32), pad]).reshape(
      TOTCH_PAD, K)

  degp = _deg_kernel(dst)
  h1p, dinv = _tc1(x, W1, degp)
  p = _agg16(src, dst, h1p)
  h2p = _tc2(p, h1p, dinv, b1.reshape(1, 16), W2)
  q = _agg32(src, dst, h2p)
  return _tc3(q, h2p, dinv, b2.reshape(1, 32), Wf1,
              bf1.reshape(1, 64), Wf2, bf2.reshape(1, 32), Wf3,
              bf3.reshape(1, 40))


# speedup vs baseline: 1.8291x; 1.0840x over previous
import functools

import jax
import jax.numpy as jnp
from jax import lax
from jax.experimental import pallas as pl
from jax.experimental.pallas import tpu as pltpu
from jax.experimental.pallas import tpu_sc as plsc

N = 10000
E = 320000
NC = 2
NS = 16
PN = 10240
K = 128
G = 5
C0 = 80
C1 = 80
CMAX = max(C0, C1)
TOTCH = NS * (C0 + C1)
TOTCH_PAD = TOTCH + CMAX
EP = TOTCH_PAD * K
RPT = PN // NS
ZR = 128


def _mesh():
  return plsc.VectorSubcoreMesh(
      core_axis_name="c", subcore_axis_name="s",
      num_cores=NC, num_subcores=NS)


def _make_edge_agg(F):
  @functools.partial(
      pl.kernel,
      out_type=jax.ShapeDtypeStruct((NC, PN, F), jnp.float32),
      mesh=_mesh(),
      compiler_params=pltpu.CompilerParams(use_tc_tiling_on_sc=False),
      scratch_types=(
          [pltpu.VMEM((CMAX, K), jnp.int32)] * 2
          + [pltpu.VMEM((K, F), jnp.float32)] * (2 * G)
          + [pltpu.VMEM((ZR, F), jnp.float32)]
          + [pltpu.VMEM_SHARED((PN, F), jnp.float32)]
          + [pltpu.VMEM_SHARED((PN, F), jnp.float32)]
          + [pltpu.SemaphoreType.DMA] * 4
      ),
  )
  def agg(src_hbm, dst_hbm, hp_hbm, out_hbm, *refs):
    sidx, didx = refs[0], refs[1]
    buf_a = refs[2:2 + G]
    buf_b = refs[2 + G:2 + 2 * G]
    zbuf = refs[2 + 2 * G]
    acc = refs[3 + 2 * G]
    hp_s = refs[4 + 2 * G]
    gs_a, gs_b, ss_a, ss_b = refs[5 + 2 * G:9 + 2 * G]

    c = lax.axis_index("c")
    s = lax.axis_index("s")
    start = jnp.where(c == 0, s * C0, NS * C0 + s * C1)
    ng = jnp.where(c == 0, C0 // G, C1 // G)
    np_ = ng // 2

    pltpu.sync_copy(src_hbm.at[pl.ds(start, CMAX)], sidx)
    pltpu.sync_copy(dst_hbm.at[pl.ds(start, CMAX)], didx)
    z = jnp.zeros((16,), jnp.float32)
    for i in range(ZR):
      for j in range(F // 16):
        zbuf[i, pl.ds(16 * j, 16)] = z
    for j in range(RPT // ZR):
      pltpu.sync_copy(zbuf, acc.at[pl.ds(s * RPT + j * ZR, ZR)])
    pltpu.sync_copy(hp_hbm.at[pl.ds(s * RPT, RPT)],
                    hp_s.at[pl.ds(s * RPT, RPT)])

    def fire_g(bufs, base, sem):
      for t in range(G):
        pltpu.async_copy(hp_s.at[sidx.at[base + t]], bufs[t], sem)

    def drain_g(bufs, sem):
      for t in range(G):
        pltpu.make_async_copy(hp_s.at[sidx.at[0]], bufs[t], sem).wait()

    def fire_s(bufs, base, sem):
      for t in range(G):
        pltpu.async_copy(bufs[t], acc.at[didx.at[base + t]], sem, add=True)

    def drain_s(bufs, sem):
      for t in range(G):
        pltpu.make_async_copy(bufs[t], acc.at[didx.at[0]], sem).wait()

    plsc.subcore_barrier()
    fire_g(buf_a, 0, gs_a)

    def pair(p, carry):
      a0 = (2 * p) * G
      b1 = (2 * p + 1) * G
      a2 = (2 * p + 2) * G
      fire_g(buf_b, b1, gs_b)
      drain_g(buf_a, gs_a)
      fire_s(buf_a, a0, ss_a)
      drain_s(buf_a, ss_a)
      fire_g(buf_a, a2, gs_a)
      drain_g(buf_b, gs_b)
      fire_s(buf_b, b1, ss_b)
      drain_s(buf_b, ss_b)
      return carry

    lax.fori_loop(0, np_ - 1, pair, 0)

    fire_g(buf_b, (ng - 1) * G, gs_b)
    drain_g(buf_a, gs_a)
    fire_s(buf_a, (ng - 2) * G, ss_a)
    drain_s(buf_a, ss_a)
    drain_g(buf_b, gs_b)
    fire_s(buf_b, (ng - 1) * G, ss_b)
    drain_s(buf_b, ss_b)

    plsc.subcore_barrier()
    pltpu.sync_copy(acc.at[pl.ds(s * RPT, RPT)],
                    out_hbm.at[c, pl.ds(s * RPT, RPT)])

  return agg


def _make_deg():
  LAG = 8

  @functools.partial(
      pl.kernel,
      out_type=jax.ShapeDtypeStruct((NC, PN, 16), jnp.float32),
      mesh=_mesh(),
      compiler_params=pltpu.CompilerParams(use_tc_tiling_on_sc=False),
      scratch_types=[
          pltpu.VMEM((CMAX, K), jnp.int32),
          pltpu.VMEM((K, 16), jnp.float32),
          pltpu.VMEM((ZR, 16), jnp.float32),
          pltpu.VMEM_SHARED((PN, 16), jnp.float32),
          pltpu.SemaphoreType.DMA,
      ],
  )
  def deg(dst_hbm, out_hbm, didx, obuf, zbuf, acc, sem):
    c = lax.axis_index("c")
    s = lax.axis_index("s")
    start = jnp.where(c == 0, s * C0, NS * C0 + s * C1)
    nch = jnp.where(c == 0, C0, C1)
    z = jnp.zeros((16,), jnp.float32)
    one = jnp.full((16,), 1.0, jnp.float32)
    for i in range(ZR):
      zbuf[i, pl.ds(0, 16)] = z
    for i in range(K):
      obuf[i, pl.ds(0, 16)] = one
    for j in range(RPT // ZR):
      pltpu.sync_copy(zbuf, acc.at[pl.ds(s * RPT + j * ZR, ZR)])
    pltpu.sync_copy(dst_hbm.at[pl.ds(start, CMAX)], didx)
    plsc.subcore_barrier()

    def chunk(i, carry):
      pltpu.async_copy(obuf, acc.at[didx.at[i]], sem, add=True)

      @pl.when(i >= LAG)
      def _():
        pltpu.make_async_copy(obuf, acc.at[didx.at[0]], sem).wait()

      return carry

    lax.fori_loop(0, nch, chunk, 0)
    for _ in range(LAG):
      pltpu.make_async_copy(obuf, acc.at[didx.at[0]], sem).wait()
    plsc.subcore_barrier()
    pltpu.sync_copy(acc.at[pl.ds(s * RPT, RPT)],
                    out_hbm.at[c, pl.ds(s * RPT, RPT)])

  return deg


_deg_kernel = _make_deg()
_agg16 = _make_edge_agg(16)


def _tc_mm1(x, W1):
  def body(x_ref, w_ref, h_ref):
    h_ref[...] = jnp.dot(x_ref[...], w_ref[...],
                         preferred_element_type=jnp.float32)

  return pl.pallas_call(
      body,
      out_shape=jax.ShapeDtypeStruct((N, 16), jnp.float32),
  )(x, W1)


def _tc1(h1, degp):
  def body(h1_ref, d_ref, h_ref, dinv_ref):
    deg = d_ref[0, pl.ds(0, N), 0:1] + d_ref[1, pl.ds(0, N), 0:1] + 1.0
    dinv = lax.rsqrt(deg)
    h_ref[pl.ds(0, N), :] = h1_ref[...] * dinv
    h_ref[pl.ds(N, PN - N), :] = jnp.zeros((PN - N, 16), jnp.float32)
    dinv_ref[...] = dinv

  return pl.pallas_call(
      body,
      out_shape=(jax.ShapeDtypeStruct((PN, 16), jnp.float32),
                 jax.ShapeDtypeStruct((N, 1), jnp.float32)),
  )(h1, degp)


def _tc2(pp, hp, dinv, b1):
  def body(p_ref, hp_ref, dinv_ref, b1_ref, out_ref):
    dinv = dinv_ref[...]
    agg = (p_ref[0, pl.ds(0, N), :] + p_ref[1, pl.ds(0, N), :]
           + hp_ref[pl.ds(0, N), :]) * dinv
    out1 = jnp.maximum(agg + b1_ref[...], 0.0)
    out_ref[pl.ds(0, N), :] = out1 * dinv
    out_ref[pl.ds(N, PN - N), :] = jnp.zeros((PN - N, 16), jnp.float32)

  return pl.pallas_call(
      body,
      out_shape=jax.ShapeDtypeStruct((PN, 16), jnp.float32),
  )(pp, hp, dinv, b1)


def _tc3(pp, hp, dinv, W2, b2, Wf1, bf1, Wf2, bf2, Wf3, bf3):
  def body(p_ref, hp_ref, dinv_ref, w2_ref, b2_ref, wf1_ref, bf1_ref,
           wf2_ref, bf2_ref, wf3_ref, bf3_ref, out_ref):
    agg = (p_ref[0, pl.ds(0, N), :] + p_ref[1, pl.ds(0, N), :]
           + hp_ref[pl.ds(0, N), :]) * dinv_ref[...]
    h2 = jnp.dot(agg, w2_ref[...], preferred_element_type=jnp.float32)
    out2 = jnp.maximum(h2 + b2_ref[...], 0.0)
    y = jnp.maximum(
        jnp.dot(out2, wf1_ref[...], preferred_element_type=jnp.float32)
        + bf1_ref[...], 0.0)
    y = jnp.maximum(
        jnp.dot(y, wf2_ref[...], preferred_element_type=jnp.float32)
        + bf2_ref[...], 0.0)
    out_ref[...] = (
        jnp.dot(y, wf3_ref[...], preferred_element_type=jnp.float32)
        + bf3_ref[...])

  return pl.pallas_call(
      body,
      out_shape=jax.ShapeDtypeStruct((N, 40), jnp.float32),
  )(pp, hp, dinv, W2, b2, Wf1, bf1, Wf2, bf2, Wf3, bf3)


def kernel(x, edge_index, W1, b1, W2, b2, Wf1, bf1, Wf2, bf2, Wf3, bf3):
  pad = jnp.full((EP - E,), N, jnp.int32)
  src = jnp.concatenate([edge_index[0].astype(jnp.int32), pad]).reshape(
      TOTCH_PAD, K)
  dst = jnp.concatenate([edge_index[1].astype(jnp.int32), pad]).reshape(
      TOTCH_PAD, K)

  h1 = _tc_mm1(x, W1)
  degp = _deg_kernel(dst)
  h1p, dinv = _tc1(h1, degp)
  p = _agg16(src, dst, h1p)
  o1p = _tc2(p, h1p, dinv, b1.reshape(1, 16))
  q = _agg16(src, dst, o1p)
  return _tc3(q, o1p, dinv, W2, b2.reshape(1, 32), Wf1,
              bf1.reshape(1, 64), Wf2, bf2.reshape(1, 32), Wf3,
              bf3.reshape(1, 40))
